# Initial kernel scaffold; baseline (speedup 1.0000x reference)
import jax
import jax.numpy as jnp
from jax.experimental import pallas as pl


def kernel(x, edge_index, W1, b1, W2, b2, gamma, beta):
    raise NotImplementedError("write your pallas kernel here")


# speedup vs baseline: 4.3517x; 4.3517x over previous
import functools

import jax
import jax.numpy as jnp
from jax import lax
from jax.experimental import pallas as pl
from jax.experimental.pallas import tpu as pltpu
from jax.experimental.pallas import tpu_sc as plsc

NC = 2
NS = 16
NW = NC * NS
K = 128


def _sc_agg_call(n_pad, e_pw, d):
    mesh = plsc.VectorSubcoreMesh(core_axis_name="c", subcore_axis_name="s")
    rows_per_tile = n_pad // NS

    @functools.partial(
        pl.kernel,
        mesh=mesh,
        out_type=jax.ShapeDtypeStruct((NC, n_pad, d), jnp.float32),
        scratch_types=[
            pltpu.VMEM((K,), jnp.int32),
            pltpu.VMEM((K,), jnp.int32),
            pltpu.VMEM((K, d), jnp.float32),
            pltpu.VMEM_SHARED((n_pad, d), jnp.float32),
            pltpu.SemaphoreType.DMA,
        ],
    )
    def sc_agg(x_hbm, src_hbm, dst_hbm, zeros_hbm, out_hbm,
               src_v, dst_v, rows_v, agg_sh, sem):
        c = lax.axis_index("c")
        s = lax.axis_index("s")
        wid = c * NS + s
        pltpu.sync_copy(
            zeros_hbm.at[pl.ds(s * rows_per_tile, rows_per_tile)],
            agg_sh.at[pl.ds(s * rows_per_tile, rows_per_tile)],
        )
        plsc.subcore_barrier()

        base = wid * e_pw

        def step(g, carry):
            off = pl.multiple_of(base + g * K, K)
            pltpu.sync_copy(src_hbm.at[pl.ds(off, K)], src_v)
            pltpu.sync_copy(dst_hbm.at[pl.ds(off, K)], dst_v)
            pltpu.async_copy(x_hbm.at[src_v], rows_v, sem).wait()
            pltpu.sync_copy(rows_v, agg_sh.at[dst_v], add=True)
            return carry

        lax.fori_loop(0, e_pw // K, step, 0)
        plsc.subcore_barrier()
        pltpu.sync_copy(
            agg_sh.at[pl.ds(s * rows_per_tile, rows_per_tile)],
            out_hbm.at[c, pl.ds(s * rows_per_tile, rows_per_tile)],
        )

    return sc_agg


def _dense_body(n, xr, p0r, p1r, w1r, b1r, w2r, b2r, gr, br, outr):
    h = xr[...] + p0r[...][:n] + p1r[...][:n]
    a = jnp.dot(h, w1r[...], preferred_element_type=jnp.float32) + b1r[...]
    a = jnp.maximum(a, 0.0)
    h2 = jnp.dot(a, w2r[...], preferred_element_type=jnp.float32) + b2r[...]
    mean = jnp.mean(h2, axis=0, keepdims=True)
    cent = h2 - mean
    var = jnp.mean(cent * cent, axis=0, keepdims=True)
    scale = lax.rsqrt(var + 1e-5) * gr[...]
    outr[...] = jnp.maximum(cent * scale + br[...], 0.0)


def kernel(x, edge_index, W1, b1, W2, b2, gamma, beta):
    n, d = x.shape
    e = edge_index.shape[1]
    e_pw = -(-e // (NW * K)) * K
    e_pad = e_pw * NW
    n_pad = -(-(n + 1) // (NS * 8)) * (NS * 8)
    dummy = n_pad - 1

    src = edge_index[0].astype(jnp.int32)
    dst = edge_index[1].astype(jnp.int32)
    src_p = jnp.concatenate([src, jnp.zeros((e_pad - e,), jnp.int32)])
    dst_p = jnp.concatenate([dst, jnp.full((e_pad - e,), dummy, jnp.int32)])
    zeros = jnp.zeros((n_pad, d), jnp.float32)

    partials = _sc_agg_call(n_pad, e_pw, d)(x, src_p, dst_p, zeros)

    out = pl.pallas_call(
        functools.partial(_dense_body, n),
        out_shape=jax.ShapeDtypeStruct((n, d), jnp.float32),
    )(x, partials[0], partials[1], W1.T, b1.reshape(1, d), W2.T,
      b2.reshape(1, d), gamma.reshape(1, d), beta.reshape(1, d))
    return out
